# Initial kernel scaffold; baseline (speedup 1.0000x reference)
import jax
import jax.numpy as jnp
from jax.experimental import pallas as pl


def kernel(timestamp, hour_table, day_table):
    raise NotImplementedError("write your pallas kernel here")


# speedup vs baseline: 3.8671x; 3.8671x over previous
import functools

import jax
import jax.numpy as jnp
from jax import lax
from jax.experimental import pallas as pl
from jax.experimental.pallas import tpu as pltpu
from jax.experimental.pallas import tpu_sc as plsc

HIDDEN = 128
TZ_SECONDS = 8 * 3600
HOURS = 24
DAYS = 7
NUM_COMBOS = HOURS * DAYS
NC, NS, LANES = 2, 16, 16
NW = NC * NS
SUB = 128


def _table_body(hour_ref, day_ref, out_ref):
    h = hour_ref[...]
    d = day_ref[...]
    out_ref[...] = (d[:, None, :] + h[None, :, :]).reshape(NUM_COMBOS, HIDDEN)


def _build_table(hour_table, day_table):
    return pl.pallas_call(
        _table_body,
        out_shape=jax.ShapeDtypeStruct((NUM_COMBOS, HIDDEN), jnp.float32),
    )(hour_table, day_table)


@functools.cache
def _make_gather(total):
    assert total % (NW * SUB) == 0
    b_per_w = total // NW
    n_sub = b_per_w // SUB

    mesh = plsc.VectorSubcoreMesh(core_axis_name="c", subcore_axis_name="s")

    @functools.partial(
        pl.kernel,
        out_type=jax.ShapeDtypeStruct((total, HIDDEN), jnp.float32),
        mesh=mesh,
        scratch_types=[
            pltpu.VMEM((b_per_w,), jnp.int32),
            pltpu.VMEM((n_sub, SUB), jnp.int32),
            pltpu.VMEM((SUB, HIDDEN), jnp.float32),
            pltpu.SemaphoreType.DMA,
        ],
    )
    def sc_kernel(ts_hbm, table_hbm, out_hbm, ts_v, idx_v, rows_v, gsem):
        wid = lax.axis_index("s") * NC + lax.axis_index("c")
        base = wid * b_per_w
        pltpu.sync_copy(ts_hbm.at[pl.ds(base, b_per_w)], ts_v)

        def idx_body(k, carry):
            j = k // (SUB // LANES)
            v = k % (SUB // LANES)
            t = ts_v[pl.ds(k * LANES, LANES)]
            e = lax.rem(lax.div(t + TZ_SECONDS, 3600), NUM_COMBOS)
            idx_v[j, pl.ds(v * LANES, LANES)] = e
            return carry

        lax.fori_loop(0, b_per_w // LANES, idx_body, 0)

        def g_body(j, carry):
            pltpu.async_copy(table_hbm.at[idx_v.at[j]], rows_v, gsem).wait()
            pltpu.sync_copy(rows_v, out_hbm.at[pl.ds(base + j * SUB, SUB)])
            return carry

        lax.fori_loop(0, n_sub, g_body, 0)

    return sc_kernel


def kernel(timestamp, hour_table, day_table):
    batch, hist = timestamp.shape
    table = _build_table(hour_table, day_table)
    ts_flat = timestamp.reshape(-1)
    out = _make_gather(batch * hist)(ts_flat, table)
    return out.reshape(batch, hist, HIDDEN)
